# Initial kernel scaffold; baseline (speedup 1.0000x reference)
import jax
import jax.numpy as jnp
from jax.experimental import pallas as pl


def kernel(x, edge_index, delta_ts, delta_ss, edge_type, Wq, Wk, Wv, Wo, time_w, time_b, dist_w, dist_b, type_table):
    raise NotImplementedError("write your pallas kernel here")


# speedup vs baseline: 12.6257x; 12.6257x over previous
import functools
import math

import jax
import jax.numpy as jnp
from jax import lax
from jax.experimental import pallas as pl
from jax.experimental.pallas import tpu as pltpu
from jax.experimental.pallas import tpu_sc as plsc

F32 = jnp.float32
_HP = lax.Precision.HIGHEST

_N = 10000
_E = 320000
_D = 128
_H = 4
_DH = 32

_NC = 2
_NS = 16
_NW = _NC * _NS
_EPW = _E // _NW
_CH = 80
_NCH = _EPW // _CH
_STRIPE = 624
_TAIL = _N - _NS * _STRIPE

_BN = 1000
_BE = 3200


def _proj_body(x_ref, wq_ref, wk_ref, wv_ref, xq_ref, tkv_ref):
    xb = x_ref[...]
    xq_ref[...] = jnp.dot(xb, wq_ref[...], preferred_element_type=F32,
                          precision=_HP)
    tkv_ref[:, 0:_D] = jnp.dot(xb, wk_ref[...], preferred_element_type=F32,
                               precision=_HP)
    tkv_ref[:, _D:2 * _D] = jnp.dot(xb, wv_ref[...],
                                    preferred_element_type=F32, precision=_HP)


def _proj(x, wq, wk, wv):
    n = x.shape[0]
    full = lambda s: pl.BlockSpec(s, lambda i: (0, 0))
    return pl.pallas_call(
        _proj_body,
        grid=(n // _BN,),
        in_specs=[pl.BlockSpec((_BN, _D), lambda i: (i, 0)),
                  full((_D, _D)), full((_D, _D)), full((_D, _D))],
        out_specs=[pl.BlockSpec((_BN, _D), lambda i: (i, 0)),
                   pl.BlockSpec((_BN, 2 * _D), lambda i: (i, 0))],
        out_shape=[jax.ShapeDtypeStruct((n, _D), F32),
                   jax.ShapeDtypeStruct((n, 2 * _D), F32)],
    )(x, wq, wk, wv)


def _gather_body(src_hbm, dst_hbm, tkv_hbm, xq_hbm, gsrc_hbm, gdst_hbm,
                 idxs, idxd, rows_s, rows_d, sem_s, sem_d):
    wid = lax.axis_index("s") * _NC + lax.axis_index("c")
    base = wid * _EPW

    def chunk(j, carry):
        off = base + j * _CH
        pltpu.sync_copy(src_hbm.at[pl.ds(off, _CH)], idxs)
        pltpu.sync_copy(dst_hbm.at[pl.ds(off, _CH)], idxd)
        cs = pltpu.async_copy(tkv_hbm.at[idxs], rows_s, sem_s)
        cd = pltpu.async_copy(xq_hbm.at[idxd], rows_d, sem_d)
        cs.wait()
        cd.wait()
        pltpu.sync_copy(rows_s, gsrc_hbm.at[pl.ds(off, _CH)])
        pltpu.sync_copy(rows_d, gdst_hbm.at[pl.ds(off, _CH)])
        return carry

    lax.fori_loop(0, _NCH, chunk, 0)


def _gather(src1d, dst1d, tkv, xq):
    mesh = plsc.VectorSubcoreMesh(core_axis_name="c", subcore_axis_name="s")
    fn = functools.partial(
        pl.kernel, mesh=mesh,
        out_type=[jax.ShapeDtypeStruct((_E, 2 * _D), F32),
                  jax.ShapeDtypeStruct((_E, _D), F32)],
        scratch_types=[pltpu.VMEM((_CH,), jnp.int32),
                       pltpu.VMEM((_CH,), jnp.int32),
                       pltpu.VMEM((_CH, 2 * _D), F32),
                       pltpu.VMEM((_CH, _D), F32),
                       pltpu.SemaphoreType.DMA,
                       pltpu.SemaphoreType.DMA],
    )(_gather_body)
    return fn(src1d, dst1d, tkv, xq)


def _edge_body(dt_ref, ds_ref, et_ref, tw_ref, tb_ref, dw_ref, db_ref,
               tt_ref, wk_ref, wv_ref, s16_ref, b16_ref, gsrc_ref, gdst_ref,
               uv_ref, ex_ref):
    dt = dt_ref[...] * (1.0 / 3600.0)
    dss = ds_ref[...]
    emb = (jnp.cos(dt * tw_ref[...] + tb_ref[...])
           + jnp.cos(dss * dw_ref[...] + db_ref[...]))
    et = et_ref[...]
    for t in range(5):
        emb = emb + jnp.where(et == t, 1.0, 0.0) * tt_ref[t:t + 1, :]
    k = gsrc_ref[:, 0:_D] + jnp.dot(emb, wk_ref[...],
                                    preferred_element_type=F32, precision=_HP)
    v = gsrc_ref[:, _D:2 * _D] + jnp.dot(emb, wv_ref[...],
                                         preferred_element_type=F32,
                                         precision=_HP)
    p = gdst_ref[...] * k
    ex = jnp.exp(jnp.dot(p, s16_ref[...], preferred_element_type=F32,
                         precision=_HP))
    uv_ref[...] = jnp.dot(ex, b16_ref[...], preferred_element_type=F32,
                          precision=_HP) * v
    ex_ref[:, 0:16] = ex
    ex_ref[:, 16:_D] = jnp.zeros_like(ex_ref[:, 16:_D])


def _edge(dt2, ds2, et2, tw2, tb2, dw2, db2, ttab, wk, wv, s16, b16,
          gsrc, gdst):
    e = dt2.shape[0]
    col = lambda: pl.BlockSpec((_BE, 1), lambda i: (i, 0))
    row = lambda: pl.BlockSpec((1, _D), lambda i: (0, 0))
    return pl.pallas_call(
        _edge_body,
        grid=(e // _BE,),
        in_specs=[col(), col(), col(), row(), row(), row(), row(),
                  pl.BlockSpec((5, _D), lambda i: (0, 0)),
                  pl.BlockSpec((_D, _D), lambda i: (0, 0)),
                  pl.BlockSpec((_D, _D), lambda i: (0, 0)),
                  pl.BlockSpec((_D, 16), lambda i: (0, 0)),
                  pl.BlockSpec((16, _D), lambda i: (0, 0)),
                  pl.BlockSpec((_BE, 2 * _D), lambda i: (i, 0)),
                  pl.BlockSpec((_BE, _D), lambda i: (i, 0))],
        out_specs=[pl.BlockSpec((_BE, _D), lambda i: (i, 0)),
                   pl.BlockSpec((_BE, _D), lambda i: (i, 0))],
        out_shape=[jax.ShapeDtypeStruct((e, _D), F32),
                   jax.ShapeDtypeStruct((e, _D), F32)],
    )(dt2, ds2, et2, tw2, tb2, dw2, db2, ttab, wk, wv, s16, b16, gsrc, gdst)


def _scatter_body(dst_hbm, val_hbm, zacc_hbm, acc2_hbm,
                  idxb, valb, acc_sh):
    cid = lax.axis_index("c")
    sid = lax.axis_index("s")
    wid = sid * _NC + cid
    base = wid * _EPW
    rs = sid * _STRIPE
    pltpu.sync_copy(zacc_hbm.at[pl.ds(rs, _STRIPE)],
                    acc_sh.at[pl.ds(rs, _STRIPE)])

    @pl.when(sid == 0)
    def _():
        pltpu.sync_copy(zacc_hbm.at[pl.ds(_NS * _STRIPE, _TAIL)],
                        acc_sh.at[pl.ds(_NS * _STRIPE, _TAIL)])

    plsc.subcore_barrier()

    def chunk(j, carry):
        off = base + j * _CH
        pltpu.sync_copy(dst_hbm.at[pl.ds(off, _CH)], idxb)
        pltpu.sync_copy(val_hbm.at[pl.ds(off, _CH)], valb)
        pltpu.sync_copy(valb, acc_sh.at[idxb], add=True)
        plsc.subcore_barrier()
        return carry

    lax.fori_loop(0, _NCH, chunk, 0)
    plsc.subcore_barrier()
    pltpu.sync_copy(acc_sh.at[pl.ds(rs, _STRIPE)],
                    acc2_hbm.at[cid, pl.ds(rs, _STRIPE)])

    @pl.when(sid == 0)
    def _():
        pltpu.sync_copy(acc_sh.at[pl.ds(_NS * _STRIPE, _TAIL)],
                        acc2_hbm.at[cid, pl.ds(_NS * _STRIPE, _TAIL)])


def _scatter(dst1d, vals, zacc):
    mesh = plsc.VectorSubcoreMesh(core_axis_name="c", subcore_axis_name="s")
    fn = functools.partial(
        pl.kernel, mesh=mesh,
        out_type=[jax.ShapeDtypeStruct((_NC, _N, _D), F32)],
        scratch_types=[pltpu.VMEM((_CH,), jnp.int32),
                       pltpu.VMEM((_CH, _D), F32),
                       pltpu.VMEM_SHARED((_N, _D), F32)],
    )(_scatter_body)
    return fn(dst1d, vals, zacc)


def _final_body(n2_ref, e2_ref, b16_ref, wo_ref, out_ref):
    num = n2_ref[0] + n2_ref[1]
    den = (e2_ref[0] + e2_ref[1])[:, 0:16]
    den_bc = jnp.dot(den, b16_ref[...], preferred_element_type=F32,
                     precision=_HP) + 1e-16
    t = jnp.dot(num / den_bc, wo_ref[...], preferred_element_type=F32,
                precision=_HP)
    out_ref[...] = jnp.where(t > 0.0, t, jnp.exp(t) - 1.0)


def _final(num2, ex2, b16, wo):
    n = num2.shape[1]
    return pl.pallas_call(
        _final_body,
        grid=(n // _BN,),
        in_specs=[pl.BlockSpec((_NC, _BN, _D), lambda i: (0, i, 0)),
                  pl.BlockSpec((_NC, _BN, _D), lambda i: (0, i, 0)),
                  pl.BlockSpec((16, _D), lambda i: (0, 0)),
                  pl.BlockSpec((_D, _D), lambda i: (0, 0))],
        out_specs=pl.BlockSpec((_BN, _D), lambda i: (i, 0)),
        out_shape=jax.ShapeDtypeStruct((n, _D), F32),
    )(num2, ex2, b16, wo)


def kernel(x, edge_index, delta_ts, delta_ss, edge_type, Wq, Wk, Wv, Wo,
           time_w, time_b, dist_w, dist_b, type_table):
    src = edge_index[0].astype(jnp.int32)
    dst = edge_index[1].astype(jnp.int32)
    et = edge_type.astype(jnp.int32)

    dt2 = delta_ts.astype(F32).reshape(_E, 1)
    ds2 = delta_ss.astype(F32).reshape(_E, 1)
    et2 = et.reshape(_E, 1)
    tw2 = time_w.astype(F32).reshape(1, _D)
    tb2 = time_b.astype(F32).reshape(1, _D)
    dw2 = dist_w.astype(F32).reshape(1, _D)
    db2 = dist_b.astype(F32).reshape(1, _D)

    hh = jnp.arange(_D, dtype=jnp.int32) // _DH
    s16 = (hh[:, None] == jnp.arange(16, dtype=jnp.int32)[None, :]
           ).astype(F32) * (1.0 / math.sqrt(_DH))
    b16 = (hh[None, :] == jnp.arange(16, dtype=jnp.int32)[:, None]
           ).astype(F32)
    zacc = jnp.zeros((_N, _D), F32)

    xq, tkv = _proj(x.astype(F32), Wq.astype(F32), Wk.astype(F32),
                    Wv.astype(F32))
    gsrc, gdst = _gather(src, dst, tkv, xq)
    uv, ex128 = _edge(dt2, ds2, et2, tw2, tb2, dw2, db2,
                      type_table.astype(F32), Wk.astype(F32), Wv.astype(F32),
                      s16, b16, gsrc, gdst)
    num2, = _scatter(dst, uv, zacc)
    ex2, = _scatter(dst, ex128, zacc)
    return _final(num2, ex2, b16, Wo.astype(F32))


# speedup vs baseline: 12.7881x; 1.0129x over previous
import functools
import math

import jax
import jax.numpy as jnp
from jax import lax
from jax.experimental import pallas as pl
from jax.experimental.pallas import tpu as pltpu
from jax.experimental.pallas import tpu_sc as plsc

F32 = jnp.float32
_HP = lax.Precision.HIGHEST

_N = 10000
_E = 320000
_D = 128
_H = 4
_DH = 32

_NC = 2
_NS = 16
_NW = _NC * _NS
_EPW = _E // _NW
_CH = 80
_NCH = _EPW // _CH
_STRIPE = 624
_TAIL = _N - _NS * _STRIPE

_BN = 1000
_BE = 3200


def _proj_body(x_ref, wq_ref, wk_ref, wv_ref, xq_ref, tkv_ref):
    xb = x_ref[...]
    xq_ref[...] = jnp.dot(xb, wq_ref[...], preferred_element_type=F32,
                          precision=_HP)
    tkv_ref[:, 0:_D] = jnp.dot(xb, wk_ref[...], preferred_element_type=F32,
                               precision=_HP)
    tkv_ref[:, _D:2 * _D] = jnp.dot(xb, wv_ref[...],
                                    preferred_element_type=F32, precision=_HP)


def _proj(x, wq, wk, wv):
    n = x.shape[0]
    full = lambda s: pl.BlockSpec(s, lambda i: (0, 0))
    return pl.pallas_call(
        _proj_body,
        grid=(n // _BN,),
        in_specs=[pl.BlockSpec((_BN, _D), lambda i: (i, 0)),
                  full((_D, _D)), full((_D, _D)), full((_D, _D))],
        out_specs=[pl.BlockSpec((_BN, _D), lambda i: (i, 0)),
                   pl.BlockSpec((_BN, 2 * _D), lambda i: (i, 0))],
        out_shape=[jax.ShapeDtypeStruct((n, _D), F32),
                   jax.ShapeDtypeStruct((n, 2 * _D), F32)],
    )(x, wq, wk, wv)


def _gather_body(src_hbm, dst_hbm, tkv_hbm, xq_hbm, gsrc_hbm, gdst_hbm,
                 idxs, idxd, rows_s, rows_d, sem_s, sem_d):
    wid = lax.axis_index("s") * _NC + lax.axis_index("c")
    base = wid * _EPW

    def chunk(j, carry):
        off = base + j * _CH
        pltpu.sync_copy(src_hbm.at[pl.ds(off, _CH)], idxs)
        pltpu.sync_copy(dst_hbm.at[pl.ds(off, _CH)], idxd)
        cs = pltpu.async_copy(tkv_hbm.at[idxs], rows_s, sem_s)
        cd = pltpu.async_copy(xq_hbm.at[idxd], rows_d, sem_d)
        cs.wait()
        cd.wait()
        pltpu.sync_copy(rows_s, gsrc_hbm.at[pl.ds(off, _CH)])
        pltpu.sync_copy(rows_d, gdst_hbm.at[pl.ds(off, _CH)])
        return carry

    lax.fori_loop(0, _NCH, chunk, 0)


def _gather(src1d, dst1d, tkv, xq):
    mesh = plsc.VectorSubcoreMesh(core_axis_name="c", subcore_axis_name="s")
    fn = functools.partial(
        pl.kernel, mesh=mesh,
        out_type=[jax.ShapeDtypeStruct((_E, 2 * _D), F32),
                  jax.ShapeDtypeStruct((_E, _D), F32)],
        scratch_types=[pltpu.VMEM((_CH,), jnp.int32),
                       pltpu.VMEM((_CH,), jnp.int32),
                       pltpu.VMEM((_CH, 2 * _D), F32),
                       pltpu.VMEM((_CH, _D), F32),
                       pltpu.SemaphoreType.DMA,
                       pltpu.SemaphoreType.DMA],
    )(_gather_body)
    return fn(src1d, dst1d, tkv, xq)


def _edge_body(dt_ref, ds_ref, et_ref, tw_ref, tb_ref, dw_ref, db_ref,
               tt_ref, wk_ref, wv_ref, s16_ref, b16_ref, gsrc_ref, gdst_ref,
               uv_ref, ex_ref):
    dt = dt_ref[...] * (1.0 / 3600.0)
    dss = ds_ref[...]
    emb = (jnp.cos(dt * tw_ref[...] + tb_ref[...])
           + jnp.cos(dss * dw_ref[...] + db_ref[...]))
    et = et_ref[...]
    for t in range(5):
        emb = emb + jnp.where(et == t, 1.0, 0.0) * tt_ref[t:t + 1, :]
    k = gsrc_ref[:, 0:_D] + jnp.dot(emb, wk_ref[...],
                                    preferred_element_type=F32, precision=_HP)
    v = gsrc_ref[:, _D:2 * _D] + jnp.dot(emb, wv_ref[...],
                                         preferred_element_type=F32,
                                         precision=_HP)
    p = gdst_ref[...] * k
    ex = jnp.exp(jnp.dot(p, s16_ref[...], preferred_element_type=F32,
                         precision=_HP))
    uv_ref[...] = jnp.dot(ex, b16_ref[...], preferred_element_type=F32,
                          precision=_HP) * v
    ex_ref[:, 0:16] = ex
    ex_ref[:, 16:_D] = jnp.zeros_like(ex_ref[:, 16:_D])


def _edge(dt2, ds2, et2, tw2, tb2, dw2, db2, ttab, wk, wv, s16, b16,
          gsrc, gdst):
    e = dt2.shape[0]
    col = lambda: pl.BlockSpec((_BE, 1), lambda i: (i, 0))
    row = lambda: pl.BlockSpec((1, _D), lambda i: (0, 0))
    return pl.pallas_call(
        _edge_body,
        grid=(e // _BE,),
        in_specs=[col(), col(), col(), row(), row(), row(), row(),
                  pl.BlockSpec((5, _D), lambda i: (0, 0)),
                  pl.BlockSpec((_D, _D), lambda i: (0, 0)),
                  pl.BlockSpec((_D, _D), lambda i: (0, 0)),
                  pl.BlockSpec((_D, 16), lambda i: (0, 0)),
                  pl.BlockSpec((16, _D), lambda i: (0, 0)),
                  pl.BlockSpec((_BE, 2 * _D), lambda i: (i, 0)),
                  pl.BlockSpec((_BE, _D), lambda i: (i, 0))],
        out_specs=[pl.BlockSpec((_BE, _D), lambda i: (i, 0)),
                   pl.BlockSpec((_BE, _D), lambda i: (i, 0))],
        out_shape=[jax.ShapeDtypeStruct((e, _D), F32),
                   jax.ShapeDtypeStruct((e, _D), F32)],
    )(dt2, ds2, et2, tw2, tb2, dw2, db2, ttab, wk, wv, s16, b16, gsrc, gdst)


def _scatter_body(dst_hbm, val_hbm, zacc_hbm, acc2_hbm,
                  idxb, valb, acc_sh):
    cid = lax.axis_index("c")
    sid = lax.axis_index("s")
    wid = sid * _NC + cid
    base = wid * _EPW
    rs = sid * _STRIPE
    pltpu.sync_copy(zacc_hbm.at[pl.ds(rs, _STRIPE)],
                    acc_sh.at[pl.ds(rs, _STRIPE)])

    @pl.when(sid == 0)
    def _():
        pltpu.sync_copy(zacc_hbm.at[pl.ds(_NS * _STRIPE, _TAIL)],
                        acc_sh.at[pl.ds(_NS * _STRIPE, _TAIL)])

    plsc.subcore_barrier()

    def chunk(j, carry):
        off = base + j * _CH
        pltpu.sync_copy(dst_hbm.at[pl.ds(off, _CH)], idxb)
        pltpu.sync_copy(val_hbm.at[pl.ds(off, _CH)], valb)
        pltpu.sync_copy(valb, acc_sh.at[idxb], add=True)
        return carry

    lax.fori_loop(0, _NCH, chunk, 0)
    plsc.subcore_barrier()
    pltpu.sync_copy(acc_sh.at[pl.ds(rs, _STRIPE)],
                    acc2_hbm.at[cid, pl.ds(rs, _STRIPE)])

    @pl.when(sid == 0)
    def _():
        pltpu.sync_copy(acc_sh.at[pl.ds(_NS * _STRIPE, _TAIL)],
                        acc2_hbm.at[cid, pl.ds(_NS * _STRIPE, _TAIL)])


def _scatter(dst1d, vals, zacc):
    mesh = plsc.VectorSubcoreMesh(core_axis_name="c", subcore_axis_name="s")
    fn = functools.partial(
        pl.kernel, mesh=mesh,
        out_type=[jax.ShapeDtypeStruct((_NC, _N, _D), F32)],
        scratch_types=[pltpu.VMEM((_CH,), jnp.int32),
                       pltpu.VMEM((_CH, _D), F32),
                       pltpu.VMEM_SHARED((_N, _D), F32)],
    )(_scatter_body)
    return fn(dst1d, vals, zacc)


def _final_body(n2_ref, e2_ref, b16_ref, wo_ref, out_ref):
    num = n2_ref[0] + n2_ref[1]
    den = (e2_ref[0] + e2_ref[1])[:, 0:16]
    den_bc = jnp.dot(den, b16_ref[...], preferred_element_type=F32,
                     precision=_HP) + 1e-16
    t = jnp.dot(num / den_bc, wo_ref[...], preferred_element_type=F32,
                precision=_HP)
    out_ref[...] = jnp.where(t > 0.0, t, jnp.exp(t) - 1.0)


def _final(num2, ex2, b16, wo):
    n = num2.shape[1]
    return pl.pallas_call(
        _final_body,
        grid=(n // _BN,),
        in_specs=[pl.BlockSpec((_NC, _BN, _D), lambda i: (0, i, 0)),
                  pl.BlockSpec((_NC, _BN, _D), lambda i: (0, i, 0)),
                  pl.BlockSpec((16, _D), lambda i: (0, 0)),
                  pl.BlockSpec((_D, _D), lambda i: (0, 0))],
        out_specs=pl.BlockSpec((_BN, _D), lambda i: (i, 0)),
        out_shape=jax.ShapeDtypeStruct((n, _D), F32),
    )(num2, ex2, b16, wo)


def kernel(x, edge_index, delta_ts, delta_ss, edge_type, Wq, Wk, Wv, Wo,
           time_w, time_b, dist_w, dist_b, type_table):
    src = edge_index[0].astype(jnp.int32)
    dst = edge_index[1].astype(jnp.int32)
    et = edge_type.astype(jnp.int32)

    dt2 = delta_ts.astype(F32).reshape(_E, 1)
    ds2 = delta_ss.astype(F32).reshape(_E, 1)
    et2 = et.reshape(_E, 1)
    tw2 = time_w.astype(F32).reshape(1, _D)
    tb2 = time_b.astype(F32).reshape(1, _D)
    dw2 = dist_w.astype(F32).reshape(1, _D)
    db2 = dist_b.astype(F32).reshape(1, _D)

    hh = jnp.arange(_D, dtype=jnp.int32) // _DH
    s16 = (hh[:, None] == jnp.arange(16, dtype=jnp.int32)[None, :]
           ).astype(F32) * (1.0 / math.sqrt(_DH))
    b16 = (hh[None, :] == jnp.arange(16, dtype=jnp.int32)[:, None]
           ).astype(F32)
    zacc = jnp.zeros((_N, _D), F32)

    xq, tkv = _proj(x.astype(F32), Wq.astype(F32), Wk.astype(F32),
                    Wv.astype(F32))
    gsrc, gdst = _gather(src, dst, tkv, xq)
    uv, ex128 = _edge(dt2, ds2, et2, tw2, tb2, dw2, db2,
                      type_table.astype(F32), Wk.astype(F32), Wv.astype(F32),
                      s16, b16, gsrc, gdst)
    num2, = _scatter(dst, uv, zacc)
    ex2, = _scatter(dst, ex128, zacc)
    return _final(num2, ex2, b16, Wo.astype(F32))
